# baseline (speedup 1.0000x reference)
import functools

import jax
import jax.numpy as jnp
from jax import lax
from jax.experimental import pallas as pl
from jax.experimental.pallas import tpu as pltpu
from jax.experimental.pallas import tpu_sc as plsc

NUM_ROWS = 100000
D = 64
DP = 128
BATCH = 4096
HIST = 200
NUM_WORKERS = 32
BSTRIPE = BATCH // NUM_WORKERS
L = 16
NGRP = BSTRIPE // L
DGRP = 4


def _sc_gather_t(idx_t, table_pad):
    mesh = plsc.VectorSubcoreMesh(core_axis_name="c", subcore_axis_name="s")

    @functools.partial(
        pl.kernel,
        mesh=mesh,
        out_type=jax.ShapeDtypeStruct((HIST, D, BATCH), jnp.float32),
        compiler_params=pltpu.CompilerParams(
            use_tc_tiling_on_sc=True, needs_layout_passes=False),
        scratch_types=(
            [pltpu.VMEM((BSTRIPE,), jnp.int32) for _ in range(2)]
            + [pltpu.VMEM((BSTRIPE, DP), jnp.float32) for _ in range(2)]
            + [pltpu.VMEM((D, BSTRIPE), jnp.float32) for _ in range(2)]
            + [pltpu.SemaphoreType.DMA for _ in range(6)]
        ),
    )
    def k(idx_ref, table_ref, out_ref, *scratch):
        ir = scratch[0:2]
        rows = scratch[2:4]
        trans = scratch[4:6]
        isem = scratch[6:8]
        gsem = scratch[8:10]
        ssem = scratch[10:12]
        wid = lax.axis_index("s") * 2 + lax.axis_index("c")
        b0 = wid * BSTRIPE
        iota = lax.iota(jnp.int32, L)
        rowv = [bg * L + iota for bg in range(NGRP)]

        def idx_start(o, h):
            pltpu.async_copy(idx_ref.at[h, pl.ds(b0, BSTRIPE)], ir[o],
                             isem[o])

        def idx_wait(o):
            pltpu.make_async_copy(idx_ref.at[0, pl.ds(b0, BSTRIPE)], ir[o],
                                  isem[o]).wait()

        def gather_start(o):
            pltpu.async_copy(table_ref.at[ir[o]], rows[o], gsem[o])

        def gather_wait(o):
            pltpu.make_async_copy(table_ref.at[ir[o]], rows[o],
                                  gsem[o]).wait()

        def store_start(o, h):
            pltpu.async_copy(trans[o], out_ref.at[h, :, pl.ds(b0, BSTRIPE)],
                             ssem[o])

        def store_wait(o):
            pltpu.make_async_copy(trans[o],
                                  out_ref.at[0, :, pl.ds(b0, BSTRIPE)],
                                  ssem[o]).wait()

        def transpose(rows_v, trans_v):
            def trow(i, c):
                d0 = i * DGRP
                vals = []
                for dd in range(DGRP):
                    col = jnp.full((L,), d0 + dd, jnp.int32)
                    for bg in range(NGRP):
                        vals.append(plsc.load_gather(rows_v,
                                                     [rowv[bg], col]))
                for dd in range(DGRP):
                    for bg in range(NGRP):
                        trans_v[d0 + dd, pl.ds(bg * L, L)] = (
                            vals[dd * NGRP + bg])
                return c

            lax.fori_loop(0, D // DGRP, trow, 0)

        pltpu.sync_copy(idx_ref.at[0, pl.ds(b0, BSTRIPE)], ir[0])
        pltpu.sync_copy(idx_ref.at[1, pl.ds(b0, BSTRIPE)], ir[1])
        gather_start(0)

        def round_body(r, carry):
            for o in range(2):
                h = 2 * r + o
                @pl.when(h + 1 < HIST)
                def _():
                    gather_start(1 - o)

                gather_wait(o)

                @pl.when(h + 2 < HIST)
                def _():
                    idx_start(o, h + 2)

                @pl.when(h >= 2)
                def _():
                    store_wait(o)

                transpose(rows[o], trans[o])
                store_start(o, h)

                @pl.when(h + 2 < HIST)
                def _():
                    idx_wait(o)

            return carry

        lax.fori_loop(0, HIST // 2, round_body, 0)
        store_wait(0)
        store_wait(1)

    return k(idx_t, table_pad)


def kernel(indices, atom_embedding):
    idx_t = indices.astype(jnp.int32).T
    table_pad = jnp.pad(atom_embedding, ((0, 0), (0, DP - D)))
    out_t = _sc_gather_t(idx_t, table_pad)
    return jnp.transpose(out_t, (2, 0, 1))


# speedup vs baseline: 1.4527x; 1.4527x over previous
import functools

import jax
import jax.numpy as jnp
from jax import lax
from jax.experimental import pallas as pl
from jax.experimental.pallas import tpu as pltpu
from jax.experimental.pallas import tpu_sc as plsc

NUM_ROWS = 100000
D = 64
BATCH = 4096
HIST = 200
TOTAL = BATCH * HIST
NUM_WORKERS = 32
PER_W = TOTAL // NUM_WORKERS
CHUNK = 128
NCHUNK = PER_W // CHUNK
NBUF = 4
ROUNDS = NCHUNK // NBUF


def _sc_gather(idx_hbm, table_hbm):
    mesh = plsc.VectorSubcoreMesh(core_axis_name="c", subcore_axis_name="s")

    @functools.partial(
        pl.kernel,
        mesh=mesh,
        out_type=jax.ShapeDtypeStruct((TOTAL, D), jnp.float32),
        compiler_params=pltpu.CompilerParams(use_tc_tiling_on_sc=False),
        scratch_types=(
            [pltpu.VMEM((NCHUNK, CHUNK), jnp.int32)]
            + [pltpu.VMEM((CHUNK, D), jnp.float32) for _ in range(NBUF)]
            + [pltpu.SemaphoreType.DMA for _ in range(2 * NBUF)]
        ),
    )
    def k(idx_ref, table_ref, out_ref, idx_v, *bufs_and_sems):
        rows = bufs_and_sems[:NBUF]
        gsem = bufs_and_sems[NBUF:2 * NBUF]
        ssem = bufs_and_sems[2 * NBUF:]
        wid = lax.axis_index("s") * 2 + lax.axis_index("c")
        base = wid * PER_W
        pltpu.sync_copy(idx_ref.at[wid], idx_v)

        def gather_start(b, j):
            pltpu.async_copy(table_ref.at[idx_v.at[j]], rows[b], gsem[b])

        def gather_wait(b):
            pltpu.make_async_copy(table_ref.at[idx_v.at[0]], rows[b],
                                  gsem[b]).wait()

        def store_start(b, j):
            pltpu.async_copy(rows[b],
                             out_ref.at[pl.ds(base + j * CHUNK, CHUNK)],
                             ssem[b])

        def store_wait(b):
            pltpu.make_async_copy(rows[b],
                                  out_ref.at[pl.ds(base, CHUNK)],
                                  ssem[b]).wait()

        for b in range(NBUF):
            gather_start(b, b)

        def round_body(r, carry):
            for b in range(NBUF):
                gather_wait(b)
                store_start(b, r * NBUF + b)

            @pl.when(r < ROUNDS - 1)
            def _():
                for b in range(NBUF):
                    store_wait(b)
                    gather_start(b, (r + 1) * NBUF + b)

            return carry

        lax.fori_loop(0, ROUNDS, round_body, 0)
        for b in range(NBUF):
            store_wait(b)

    return k(idx_hbm, table_hbm)


def kernel(indices, atom_embedding):
    idx = indices.astype(jnp.int32).reshape(NUM_WORKERS, NCHUNK, CHUNK)
    out = _sc_gather(idx, atom_embedding)
    return out.reshape(BATCH, HIST, D)
